# baseline (speedup 1.0000x reference)
import functools

import jax
import jax.numpy as jnp
from jax import lax
from jax.experimental import pallas as pl
from jax.experimental.pallas import tpu as pltpu
from jax.experimental.pallas import tpu_sc as plsc

N = 10000
E = 320000
D_IN = 128
D_OUT = 128
R = 16
B = 8

NC = 2
NS = 16
NW = NC * NS
CH = 128
EPW = E // NW
PKW = 80
EPWP = PKW * CH
EP = NW * EPWP
HPK = PKW // 2
NPAD = 10240
RPT = NPAD // NS

NB = 400


def _transform_body(x_ref, w_ref, wc_ref, h_ref):
    w = w_ref[...].reshape(B, D_IN * D_OUT)
    wc = wc_ref[...]
    big_w = jnp.dot(wc, w, preferred_element_type=jnp.float32)
    big_w = big_w.reshape(R, D_IN, D_OUT).astype(jnp.bfloat16)
    xb = x_ref[...].astype(jnp.bfloat16)
    for r in range(R):
        h_ref[r] = jnp.dot(xb, big_w[r], preferred_element_type=jnp.float32)


def _combine_body(p_ref, x_ref, lw_ref, b_ref, o_ref):
    loop = jnp.dot(x_ref[...], lw_ref[...], preferred_element_type=jnp.float32)
    o_ref[...] = p_ref[0] + p_ref[1] + loop + b_ref[0]


def _sc_gather_scatter(hflat, gidx, dstidx, partial,
                       gb0, db0, gb1, db1, gb2, db2, gb3, db3,
                       rows0, rows1, acc,
                       si0, si1, si2, si3, sg0, sg1):
    c = lax.axis_index("c")
    s = lax.axis_index("s")
    w = s * NC + c
    row0 = pl.multiple_of(s * RPT, RPT)

    gbufs = (gb0, gb1, gb2, gb3)
    dbufs = (db0, db1, db2, db3)
    isems = (si0, si1, si2, si3)
    rbufs = (rows0, rows1)
    gsems = (sg0, sg1)

    zeros16 = jnp.zeros((16,), jnp.float32)

    def zrow(i, carry):
        for j in range(D_OUT // 16):
            rows0[i, pl.ds(j * 16, 16)] = zeros16
        return carry

    lax.fori_loop(0, CH, zrow, 0)
    for t in range(RPT // CH):
        pltpu.sync_copy(rows0, acc.at[pl.ds(row0 + t * CH, CH)])
    plsc.subcore_barrier()

    ebase = pl.multiple_of(w * EPWP, EPWP)

    def load_idx(k, t):
        off = pl.multiple_of(ebase + k * CH, CH)
        pltpu.make_async_copy(gidx.at[pl.ds(off, CH)], gbufs[t], isems[t]).start()
        pltpu.make_async_copy(dstidx.at[pl.ds(off, CH)], dbufs[t], isems[t]).start()

    def wait_idx(t):
        pltpu.make_async_copy(gidx.at[pl.ds(0, CH)], gbufs[t], isems[t]).wait()
        pltpu.make_async_copy(dstidx.at[pl.ds(0, CH)], dbufs[t], isems[t]).wait()

    def start_gather(t, r):
        pltpu.make_async_copy(hflat.at[gbufs[t]], rbufs[r], gsems[r]).start()

    def wait_gather(r):
        pltpu.make_async_copy(hflat.at[gbufs[0]], rbufs[r], gsems[r]).wait()

    load_idx(0, 0)
    load_idx(1, 1)
    load_idx(2, 2)
    load_idx(3, 3)
    wait_idx(0)
    start_gather(0, 0)
    wait_idx(1)
    start_gather(1, 1)

    def body(j, carry):
        for t in range(4):
            k = 4 * j + t
            wait_gather(t % 2)
            pltpu.sync_copy(rbufs[t % 2], acc.at[dbufs[t]], add=True)

            @pl.when(k < PKW - 4)
            def _():
                load_idx(k + 4, t)

            @pl.when(k < PKW - 2)
            def _():
                wait_idx((t + 2) % 4)
                start_gather((t + 2) % 4, t % 2)

        return carry

    lax.fori_loop(0, PKW // 4, body, 0)
    plsc.subcore_barrier()

    pltpu.sync_copy(acc.at[pl.ds(row0, RPT)], partial.at[c, pl.ds(row0, RPT)])


def kernel(x, edge_index, etypes, weight, w_comp, loop_weight, h_bias):
    src = edge_index[0]
    dst = edge_index[1]
    gidx = etypes * jnp.int32(N) + src

    h_all = pl.pallas_call(
        _transform_body,
        grid=(N // NB,),
        in_specs=[
            pl.BlockSpec((NB, D_IN), lambda i: (i, 0)),
            pl.BlockSpec((B, D_IN, D_OUT), lambda i: (0, 0, 0)),
            pl.BlockSpec((R, B), lambda i: (0, 0)),
        ],
        out_specs=pl.BlockSpec((R, NB, D_OUT), lambda i: (0, i, 0)),
        out_shape=jax.ShapeDtypeStruct((R, N, D_OUT), jnp.float32),
    )(x, weight, w_comp)
    hflat = h_all.reshape(R * N, D_OUT)

    npadedge = EPWP - EPW
    pad_gather = (jnp.arange(NW, dtype=jnp.int32)[:, None] * 4099
                  + jnp.arange(npadedge, dtype=jnp.int32)[None, :] * 17) % (R * N)
    gidx_p = jnp.concatenate([gidx.reshape(NW, EPW), pad_gather], axis=1) \
        .reshape(EP)
    dump_rows = jnp.broadcast_to(jnp.arange(N, N + npadedge, dtype=jnp.int32),
                                 (NW, npadedge))
    dst_p = jnp.concatenate([dst.reshape(NW, EPW), dump_rows], axis=1) \
        .reshape(EP)

    mesh = plsc.VectorSubcoreMesh(
        core_axis_name="c", subcore_axis_name="s", num_cores=NC, num_subcores=NS)
    partial = pl.kernel(
        _sc_gather_scatter,
        out_type=jax.ShapeDtypeStruct((NC, NPAD, D_OUT), jnp.float32),
        mesh=mesh,
        scratch_types=(
            [pltpu.VMEM((CH,), jnp.int32)] * 8
            + [pltpu.VMEM((CH, D_OUT), jnp.float32)] * 2
            + [pltpu.VMEM_SHARED((NPAD, D_OUT), jnp.float32)]
            + [pltpu.SemaphoreType.DMA] * 6
        ),
    )(hflat, gidx_p, dst_p)

    bias8 = jnp.broadcast_to(h_bias, (8, D_OUT))
    out = pl.pallas_call(
        _combine_body,
        grid=(N // NB,),
        in_specs=[
            pl.BlockSpec((NC, NB, D_OUT), lambda i: (0, i, 0)),
            pl.BlockSpec((NB, D_IN), lambda i: (i, 0)),
            pl.BlockSpec((D_IN, D_OUT), lambda i: (0, 0)),
            pl.BlockSpec((8, D_OUT), lambda i: (0, 0)),
        ],
        out_specs=pl.BlockSpec((NB, D_OUT), lambda i: (i, 0)),
        out_shape=jax.ShapeDtypeStruct((N, D_OUT), jnp.float32),
    )(partial, x, loop_weight, bias8)
    return out


# speedup vs baseline: 1.0847x; 1.0847x over previous
import functools

import jax
import jax.numpy as jnp
from jax import lax
from jax.experimental import pallas as pl
from jax.experimental.pallas import tpu as pltpu
from jax.experimental.pallas import tpu_sc as plsc

N = 10000
E = 320000
D_IN = 128
D_OUT = 128
R = 16
B = 8

NC = 2
NS = 16
NW = NC * NS
CH = 128
EPW = E // NW
PKW = 80
EPWP = PKW * CH
EP = NW * EPWP
HPK = PKW // 2
NPAD = 10240
RPT = NPAD // NS

NB = 1000


def _transform_body(x_ref, w_ref, wc_ref, h_ref):
    w = w_ref[...].reshape(B, D_IN * D_OUT)
    wc = wc_ref[...]
    big_w = jnp.dot(wc, w, preferred_element_type=jnp.float32)
    big_w = big_w.reshape(R, D_IN, D_OUT)
    xb = x_ref[...]
    for r in range(R):
        h_ref[r] = jnp.dot(xb, big_w[r], preferred_element_type=jnp.float32)


def _combine_body(p_ref, x_ref, lw_ref, b_ref, o_ref):
    loop = jnp.dot(x_ref[...], lw_ref[...], preferred_element_type=jnp.float32)
    o_ref[...] = p_ref[0] + p_ref[1] + loop + b_ref[0]


def _sc_gather_scatter(hflat, gidx, dstidx, partial,
                       gb0, db0, gb1, db1, gb2, db2, gb3, db3,
                       rows0, rows1, acc,
                       si0, si1, si2, si3, sg0, sg1):
    c = lax.axis_index("c")
    s = lax.axis_index("s")
    w = s * NC + c
    row0 = pl.multiple_of(s * RPT, RPT)

    gbufs = (gb0, gb1, gb2, gb3)
    dbufs = (db0, db1, db2, db3)
    isems = (si0, si1, si2, si3)
    rbufs = (rows0, rows1)
    gsems = (sg0, sg1)

    zeros16 = jnp.zeros((16,), jnp.float32)

    def zrow(i, carry):
        for j in range(D_OUT // 16):
            rows0[i, pl.ds(j * 16, 16)] = zeros16
        return carry

    lax.fori_loop(0, CH, zrow, 0)
    for t in range(RPT // CH):
        pltpu.sync_copy(rows0, acc.at[pl.ds(row0 + t * CH, CH)])
    plsc.subcore_barrier()

    ebase = pl.multiple_of(w * EPWP, EPWP)

    def load_idx(k, t):
        off = pl.multiple_of(ebase + k * CH, CH)
        pltpu.make_async_copy(gidx.at[pl.ds(off, CH)], gbufs[t], isems[t]).start()
        pltpu.make_async_copy(dstidx.at[pl.ds(off, CH)], dbufs[t], isems[t]).start()

    def wait_idx(t):
        pltpu.make_async_copy(gidx.at[pl.ds(0, CH)], gbufs[t], isems[t]).wait()
        pltpu.make_async_copy(dstidx.at[pl.ds(0, CH)], dbufs[t], isems[t]).wait()

    def start_gather(t, r):
        pltpu.make_async_copy(hflat.at[gbufs[t]], rbufs[r], gsems[r]).start()

    def wait_gather(r):
        pltpu.make_async_copy(hflat.at[gbufs[0]], rbufs[r], gsems[r]).wait()

    load_idx(0, 0)
    load_idx(1, 1)
    load_idx(2, 2)
    load_idx(3, 3)
    wait_idx(0)
    start_gather(0, 0)
    wait_idx(1)
    start_gather(1, 1)

    def body(j, carry):
        for t in range(4):
            k = 4 * j + t
            wait_gather(t % 2)
            pltpu.sync_copy(rbufs[t % 2], acc.at[dbufs[t]], add=True)

            @pl.when(k < PKW - 4)
            def _():
                load_idx(k + 4, t)

            @pl.when(k < PKW - 2)
            def _():
                wait_idx((t + 2) % 4)
                start_gather((t + 2) % 4, t % 2)

        return carry

    lax.fori_loop(0, PKW // 4, body, 0)
    plsc.subcore_barrier()

    pltpu.sync_copy(acc.at[pl.ds(row0, RPT)], partial.at[c, pl.ds(row0, RPT)])


def kernel(x, edge_index, etypes, weight, w_comp, loop_weight, h_bias):
    src = edge_index[0]
    dst = edge_index[1]
    gidx = etypes * jnp.int32(N) + src

    h_all = pl.pallas_call(
        _transform_body,
        grid=(N // NB,),
        in_specs=[
            pl.BlockSpec((NB, D_IN), lambda i: (i, 0)),
            pl.BlockSpec((B, D_IN, D_OUT), lambda i: (0, 0, 0)),
            pl.BlockSpec((R, B), lambda i: (0, 0)),
        ],
        out_specs=pl.BlockSpec((R, NB, D_OUT), lambda i: (0, i, 0)),
        out_shape=jax.ShapeDtypeStruct((R, N, D_OUT), jnp.float32),
    )(x, weight, w_comp)
    hflat = h_all.reshape(R * N, D_OUT)

    npadedge = EPWP - EPW
    pad_gather = (jnp.arange(NW, dtype=jnp.int32)[:, None] * 4099
                  + jnp.arange(npadedge, dtype=jnp.int32)[None, :] * 17) % (R * N)
    gidx_p = jnp.concatenate([gidx.reshape(NW, EPW), pad_gather], axis=1) \
        .reshape(EP)
    dump_rows = jnp.broadcast_to(jnp.arange(N, N + npadedge, dtype=jnp.int32),
                                 (NW, npadedge))
    dst_p = jnp.concatenate([dst.reshape(NW, EPW), dump_rows], axis=1) \
        .reshape(EP)

    mesh = plsc.VectorSubcoreMesh(
        core_axis_name="c", subcore_axis_name="s", num_cores=NC, num_subcores=NS)
    partial = pl.kernel(
        _sc_gather_scatter,
        out_type=jax.ShapeDtypeStruct((NC, NPAD, D_OUT), jnp.float32),
        mesh=mesh,
        scratch_types=(
            [pltpu.VMEM((CH,), jnp.int32)] * 8
            + [pltpu.VMEM((CH, D_OUT), jnp.float32)] * 2
            + [pltpu.VMEM_SHARED((NPAD, D_OUT), jnp.float32)]
            + [pltpu.SemaphoreType.DMA] * 6
        ),
    )(hflat, gidx_p, dst_p)

    bias8 = jnp.broadcast_to(h_bias, (8, D_OUT))
    out = pl.pallas_call(
        _combine_body,
        grid=(N // NB,),
        in_specs=[
            pl.BlockSpec((NC, NB, D_OUT), lambda i: (0, i, 0)),
            pl.BlockSpec((NB, D_IN), lambda i: (i, 0)),
            pl.BlockSpec((D_IN, D_OUT), lambda i: (0, 0)),
            pl.BlockSpec((8, D_OUT), lambda i: (0, 0)),
        ],
        out_specs=pl.BlockSpec((NB, D_OUT), lambda i: (i, 0)),
        out_shape=jax.ShapeDtypeStruct((N, D_OUT), jnp.float32),
    )(partial, x, loop_weight, bias8)
    return out


# speedup vs baseline: 1.1206x; 1.0331x over previous
import functools

import jax
import jax.numpy as jnp
from jax import lax
from jax.experimental import pallas as pl
from jax.experimental.pallas import tpu as pltpu
from jax.experimental.pallas import tpu_sc as plsc

N = 10000
E = 320000
D_IN = 128
D_OUT = 128
R = 16
B = 8

NC = 2
NS = 16
NW = NC * NS
CH = 128
EPW = E // NW
PKW = 80
EPWP = PKW * CH
EP = NW * EPWP
HPK = PKW // 2
NPAD = 10240
RPT = NPAD // NS

NB = 2000


def _transform_body(x_ref, w_ref, wc_ref, h_ref):
    w = w_ref[...].reshape(B, D_IN * D_OUT)
    wc = wc_ref[...]
    big_w = jnp.dot(wc, w, preferred_element_type=jnp.float32)
    big_w = big_w.reshape(R, D_IN, D_OUT)
    xb = x_ref[...]
    for r in range(R):
        h_ref[r] = jnp.dot(xb, big_w[r], preferred_element_type=jnp.float32)


def _combine_body(p_ref, x_ref, lw_ref, b_ref, o_ref):
    loop = jnp.dot(x_ref[...], lw_ref[...], preferred_element_type=jnp.float32)
    o_ref[...] = p_ref[0] + p_ref[1] + loop + b_ref[0]


def _sc_gather_scatter(hflat, gidx, dstidx, partial,
                       gb0, db0, gb1, db1, gb2, db2, gb3, db3,
                       rows0, rows1, acc,
                       si0, si1, si2, si3, sg0, sg1):
    c = lax.axis_index("c")
    s = lax.axis_index("s")
    w = s * NC + c
    row0 = pl.multiple_of(s * RPT, RPT)

    gbufs = (gb0, gb1, gb2, gb3)
    dbufs = (db0, db1, db2, db3)
    isems = (si0, si1, si2, si3)
    rbufs = (rows0, rows1)
    gsems = (sg0, sg1)

    zeros16 = jnp.zeros((16,), jnp.float32)

    def zrow(i, carry):
        for j in range(D_OUT // 16):
            rows0[i, pl.ds(j * 16, 16)] = zeros16
        return carry

    lax.fori_loop(0, CH, zrow, 0)
    for t in range(RPT // CH):
        pltpu.sync_copy(rows0, acc.at[pl.ds(row0 + t * CH, CH)])
    plsc.subcore_barrier()

    ebase = pl.multiple_of(w * EPWP, EPWP)

    def load_idx(k, t):
        off = pl.multiple_of(ebase + k * CH, CH)
        pltpu.make_async_copy(gidx.at[pl.ds(off, CH)], gbufs[t], isems[t]).start()
        pltpu.make_async_copy(dstidx.at[pl.ds(off, CH)], dbufs[t], isems[t]).start()

    def wait_idx(t):
        pltpu.make_async_copy(gidx.at[pl.ds(0, CH)], gbufs[t], isems[t]).wait()
        pltpu.make_async_copy(dstidx.at[pl.ds(0, CH)], dbufs[t], isems[t]).wait()

    def start_gather(t, r):
        pltpu.make_async_copy(hflat.at[gbufs[t]], rbufs[r], gsems[r]).start()

    def wait_gather(r):
        pltpu.make_async_copy(hflat.at[gbufs[0]], rbufs[r], gsems[r]).wait()

    load_idx(0, 0)
    load_idx(1, 1)
    load_idx(2, 2)
    load_idx(3, 3)
    wait_idx(0)
    start_gather(0, 0)
    wait_idx(1)
    start_gather(1, 1)

    def body(j, carry):
        for t in range(4):
            k = 4 * j + t
            wait_gather(t % 2)
            pltpu.sync_copy(rbufs[t % 2], acc.at[dbufs[t]], add=True)

            @pl.when(k < PKW - 4)
            def _():
                load_idx(k + 4, t)

            @pl.when(k < PKW - 2)
            def _():
                wait_idx((t + 2) % 4)
                start_gather((t + 2) % 4, t % 2)

        return carry

    lax.fori_loop(0, PKW // 4, body, 0)
    plsc.subcore_barrier()

    pltpu.sync_copy(acc.at[pl.ds(row0, RPT)], partial.at[c, pl.ds(row0, RPT)])


def kernel(x, edge_index, etypes, weight, w_comp, loop_weight, h_bias):
    src = edge_index[0]
    dst = edge_index[1]
    gidx = etypes * jnp.int32(N) + src

    h_all = pl.pallas_call(
        _transform_body,
        grid=(N // NB,),
        in_specs=[
            pl.BlockSpec((NB, D_IN), lambda i: (i, 0)),
            pl.BlockSpec((B, D_IN, D_OUT), lambda i: (0, 0, 0)),
            pl.BlockSpec((R, B), lambda i: (0, 0)),
        ],
        out_specs=pl.BlockSpec((R, NB, D_OUT), lambda i: (0, i, 0)),
        out_shape=jax.ShapeDtypeStruct((R, N, D_OUT), jnp.float32),
    )(x, weight, w_comp)
    hflat = h_all.reshape(R * N, D_OUT)

    npadedge = EPWP - EPW
    pad_gather = (jnp.arange(NW, dtype=jnp.int32)[:, None] * 4099
                  + jnp.arange(npadedge, dtype=jnp.int32)[None, :] * 17) % (R * N)
    gidx_p = jnp.concatenate([gidx.reshape(NW, EPW), pad_gather], axis=1) \
        .reshape(EP)
    dump_rows = jnp.broadcast_to(jnp.arange(N, N + npadedge, dtype=jnp.int32),
                                 (NW, npadedge))
    dst_p = jnp.concatenate([dst.reshape(NW, EPW), dump_rows], axis=1) \
        .reshape(EP)

    mesh = plsc.VectorSubcoreMesh(
        core_axis_name="c", subcore_axis_name="s", num_cores=NC, num_subcores=NS)
    partial = pl.kernel(
        _sc_gather_scatter,
        out_type=jax.ShapeDtypeStruct((NC, NPAD, D_OUT), jnp.float32),
        mesh=mesh,
        scratch_types=(
            [pltpu.VMEM((CH,), jnp.int32)] * 8
            + [pltpu.VMEM((CH, D_OUT), jnp.float32)] * 2
            + [pltpu.VMEM_SHARED((NPAD, D_OUT), jnp.float32)]
            + [pltpu.SemaphoreType.DMA] * 6
        ),
    )(hflat, gidx_p, dst_p)

    bias8 = jnp.broadcast_to(h_bias, (8, D_OUT))
    out = pl.pallas_call(
        _combine_body,
        grid=(N // NB,),
        in_specs=[
            pl.BlockSpec((NC, NB, D_OUT), lambda i: (0, i, 0)),
            pl.BlockSpec((NB, D_IN), lambda i: (i, 0)),
            pl.BlockSpec((D_IN, D_OUT), lambda i: (0, 0)),
            pl.BlockSpec((8, D_OUT), lambda i: (0, 0)),
        ],
        out_specs=pl.BlockSpec((NB, D_OUT), lambda i: (i, 0)),
        out_shape=jax.ShapeDtypeStruct((N, D_OUT), jnp.float32),
    )(partial, x, loop_weight, bias8)
    return out


# speedup vs baseline: 1.1207x; 1.0001x over previous
import jax
import jax.numpy as jnp
from jax import lax
from jax.experimental import pallas as pl
from jax.experimental.pallas import tpu as pltpu
from jax.experimental.pallas import tpu_sc as plsc

N = 10000
E = 320000
D_IN = 128
D_OUT = 128
R = 16
B = 8

NC = 2
NS = 16
NW = NC * NS
CH = 128
EPW = E // NW
PKW = 80
EPWP = PKW * CH
EP = NW * EPWP
NPAD = 10240
RPT = NPAD // NS

NB = 2000


def _transform_body(x_ref, w_ref, wc_ref, h_ref):
    w = w_ref[...].reshape(B, D_IN * D_OUT)
    wc = wc_ref[...]
    big_w = jnp.dot(wc, w, preferred_element_type=jnp.float32)
    big_w = big_w.reshape(R, D_IN, D_OUT)
    xb = x_ref[...]
    for r in range(R):
        h_ref[r] = jnp.dot(xb, big_w[r], preferred_element_type=jnp.float32)


def _combine_body(p_ref, x_ref, lw_ref, b_ref, o_ref):
    loop = jnp.dot(x_ref[...], lw_ref[...], preferred_element_type=jnp.float32)
    o_ref[...] = p_ref[0] + p_ref[1] + loop + b_ref[0]


def _sc_gather_scatter(hflat, gidx, dstidx, partial,
                       gb0, db0, gb1, db1, gb2, db2, gb3, db3,
                       rows0, rows1, acc,
                       si0, si1, si2, si3, sg0, sg1):
    c = lax.axis_index("c")
    s = lax.axis_index("s")
    w = s * NC + c
    row0 = pl.multiple_of(s * RPT, RPT)

    gbufs = (gb0, gb1, gb2, gb3)
    dbufs = (db0, db1, db2, db3)
    isems = (si0, si1, si2, si3)
    rbufs = (rows0, rows1)
    gsems = (sg0, sg1)

    zeros16 = jnp.zeros((16,), jnp.float32)

    def zrow(i, carry):
        for j in range(D_OUT // 16):
            rows0[i, pl.ds(j * 16, 16)] = zeros16
        return carry

    lax.fori_loop(0, CH, zrow, 0)
    for t in range(RPT // CH):
        pltpu.sync_copy(rows0, acc.at[pl.ds(row0 + t * CH, CH)])
    plsc.subcore_barrier()

    ebase = pl.multiple_of(w * EPWP, EPWP)

    def load_idx(k, t):
        off = pl.multiple_of(ebase + k * CH, CH)
        pltpu.make_async_copy(gidx.at[pl.ds(off, CH)], gbufs[t], isems[t]).start()
        pltpu.make_async_copy(dstidx.at[pl.ds(off, CH)], dbufs[t], isems[t]).start()

    def wait_idx(t):
        pltpu.make_async_copy(gidx.at[pl.ds(0, CH)], gbufs[t], isems[t]).wait()
        pltpu.make_async_copy(dstidx.at[pl.ds(0, CH)], dbufs[t], isems[t]).wait()

    def start_gather(t, r):
        pltpu.make_async_copy(hflat.at[gbufs[t]], rbufs[r], gsems[r]).start()

    def wait_gather(r):
        pltpu.make_async_copy(hflat.at[gbufs[0]], rbufs[r], gsems[r]).wait()

    load_idx(0, 0)
    load_idx(1, 1)
    load_idx(2, 2)
    load_idx(3, 3)
    wait_idx(0)
    start_gather(0, 0)
    wait_idx(1)
    start_gather(1, 1)

    def body(j, carry):
        for t in range(4):
            k = 4 * j + t
            wait_gather(t % 2)
            pltpu.sync_copy(rbufs[t % 2], acc.at[dbufs[t]], add=True)

            @pl.when(k < PKW - 4)
            def _():
                load_idx(k + 4, t)

            @pl.when(k < PKW - 2)
            def _():
                wait_idx((t + 2) % 4)
                start_gather((t + 2) % 4, t % 2)

        return carry

    lax.fori_loop(0, PKW // 4, body, 0)
    plsc.subcore_barrier()

    pltpu.sync_copy(acc.at[pl.ds(row0, RPT)], partial.at[c, pl.ds(row0, RPT)])


def kernel(x, edge_index, etypes, weight, w_comp, loop_weight, h_bias):
    src = edge_index[0]
    dst = edge_index[1]
    gidx = etypes * jnp.int32(N) + src

    h_all = pl.pallas_call(
        _transform_body,
        grid=(N // NB,),
        in_specs=[
            pl.BlockSpec((NB, D_IN), lambda i: (i, 0)),
            pl.BlockSpec((B, D_IN, D_OUT), lambda i: (0, 0, 0)),
            pl.BlockSpec((R, B), lambda i: (0, 0)),
        ],
        out_specs=pl.BlockSpec((R, NB, D_OUT), lambda i: (0, i, 0)),
        out_shape=jax.ShapeDtypeStruct((R, N, D_OUT), jnp.float32),
    )(x, weight, w_comp)
    hflat = h_all.reshape(R * N, D_OUT)

    npadedge = EPWP - EPW
    pad_gather = (jnp.arange(NW, dtype=jnp.int32)[:, None] * 4099
                  + jnp.arange(npadedge, dtype=jnp.int32)[None, :] * 17) % (R * N)
    gidx_p = jnp.concatenate([gidx.reshape(NW, EPW), pad_gather], axis=1) \
        .reshape(EP)
    dump_rows = jnp.broadcast_to(jnp.arange(N, N + npadedge, dtype=jnp.int32),
                                 (NW, npadedge))
    dst_p = jnp.concatenate([dst.reshape(NW, EPW), dump_rows], axis=1) \
        .reshape(EP)

    mesh = plsc.VectorSubcoreMesh(
        core_axis_name="c", subcore_axis_name="s", num_cores=NC, num_subcores=NS)
    partial = pl.kernel(
        _sc_gather_scatter,
        out_type=jax.ShapeDtypeStruct((NC, NPAD, D_OUT), jnp.float32),
        mesh=mesh,
        scratch_types=(
            [pltpu.VMEM((CH,), jnp.int32)] * 8
            + [pltpu.VMEM((CH, D_OUT), jnp.float32)] * 2
            + [pltpu.VMEM_SHARED((NPAD, D_OUT), jnp.float32)]
            + [pltpu.SemaphoreType.DMA] * 6
        ),
    )(hflat, gidx_p, dst_p)

    bias8 = jnp.broadcast_to(h_bias, (8, D_OUT))
    out = pl.pallas_call(
        _combine_body,
        grid=(N // NB,),
        in_specs=[
            pl.BlockSpec((NC, NB, D_OUT), lambda i: (0, i, 0)),
            pl.BlockSpec((NB, D_IN), lambda i: (i, 0)),
            pl.BlockSpec((D_IN, D_OUT), lambda i: (0, 0)),
            pl.BlockSpec((8, D_OUT), lambda i: (0, 0)),
        ],
        out_specs=pl.BlockSpec((NB, D_OUT), lambda i: (i, 0)),
        out_shape=jax.ShapeDtypeStruct((N, D_OUT), jnp.float32),
    )(partial, x, loop_weight, bias8)
    return out
